# baseline (speedup 1.0000x reference)
import functools

import jax
import jax.numpy as jnp
from jax import lax
from jax.experimental import pallas as pl
from jax.experimental.pallas import tpu as pltpu
from jax.experimental.pallas import tpu_sc as plsc

_NUM_CLASS = 100
_FEAT = 128
_N = 320000
_NC = 2
_NS = 16
_NW = _NC * _NS
_S = 163840
_T = _N - _S
_ROWS_PER_W = _S // _NW
_CHUNK = 128
_NFULL = _ROWS_PER_W // _CHUNK
_ACLASS = 104
_ZROWS = 8
_NBUF = 4
_TCBLK = 512


def _sc_segment_sum(batch_hbm, tgt_hbm, parts_hbm,
                    rows_buf0, rows_buf1, rows_buf2, rows_buf3,
                    tgt_buf0, tgt_buf1, tgt_buf2, tgt_buf3,
                    zero_buf, acc,
                    gsem0, gsem1, gsem2, gsem3,
                    ssem0, ssem1, ssem2, ssem3):
    cid = lax.axis_index("c")
    sid = lax.axis_index("s")
    wid = cid * _NS + sid
    base = wid * _ROWS_PER_W
    rows_bufs = (rows_buf0, rows_buf1, rows_buf2, rows_buf3)
    tgt_bufs = (tgt_buf0, tgt_buf1, tgt_buf2, tgt_buf3)
    gsems = (gsem0, gsem1, gsem2, gsem3)
    ssems = (ssem0, ssem1, ssem2, ssem3)

    zeros16 = jnp.zeros((16,), jnp.float32)

    def zero_body(i, carry):
        r = i // (_FEAT // 16)
        c = i % (_FEAT // 16)
        zero_buf[r, pl.ds(c * 16, 16)] = zeros16
        return carry

    @pl.when(sid < _ACLASS // _ZROWS)
    def _():
        lax.fori_loop(0, _ZROWS * (_FEAT // 16), zero_body, 0)
        pltpu.sync_copy(zero_buf, acc.at[pl.ds(sid * _ZROWS, _ZROWS)])

    plsc.subcore_barrier()

    def start_gather(ci, b):
        off = base + ci * _CHUNK
        pltpu.async_copy(tgt_hbm.at[pl.ds(off, _CHUNK)], tgt_bufs[b], gsems[b])
        pltpu.async_copy(batch_hbm.at[pl.ds(off, _CHUNK)], rows_bufs[b],
                         gsems[b])

    def wait_gather(ci, b):
        off = base + ci * _CHUNK
        pltpu.make_async_copy(tgt_hbm.at[pl.ds(off, _CHUNK)], tgt_bufs[b],
                              gsems[b]).wait()
        pltpu.make_async_copy(batch_hbm.at[pl.ds(off, _CHUNK)], rows_bufs[b],
                              gsems[b]).wait()

    def start_scatter(b):
        pltpu.async_copy(rows_bufs[b], acc.at[tgt_bufs[b]], ssems[b],
                         add=True)

    def wait_scatter(b):
        pltpu.make_async_copy(rows_bufs[b], acc.at[tgt_bufs[b]],
                              ssems[b]).wait()

    start_gather(0, 0)
    start_gather(1, 1)

    def chunk_body(i, carry):
        for b in range(_NBUF):
            ci = _NBUF * i + b

            @pl.when(jnp.logical_and(ci >= 2, ci + 2 < _NFULL))
            def _():
                wait_scatter((b + 2) % _NBUF)

            @pl.when(ci + 2 < _NFULL)
            def _():
                start_gather(ci + 2, (b + 2) % _NBUF)

            wait_gather(ci, b)
            start_scatter(b)
        return carry

    lax.fori_loop(0, _NFULL // _NBUF, chunk_body, 0)

    for k in range(_NBUF):
        wait_scatter((_NFULL - _NBUF + k) % _NBUF)

    plsc.subcore_barrier()

    @pl.when(sid < _ACLASS // _ZROWS)
    def _():
        pltpu.sync_copy(acc.at[pl.ds(sid * _ZROWS, _ZROWS)],
                        parts_hbm.at[cid, pl.ds(sid * _ZROWS, _ZROWS)])


_sc_call = functools.partial(
    pl.kernel,
    out_type=jax.ShapeDtypeStruct((_NC, _ACLASS, _FEAT), jnp.float32),
    mesh=plsc.VectorSubcoreMesh(core_axis_name="c", subcore_axis_name="s"),
    scratch_types=(
        [pltpu.VMEM((_CHUNK, _FEAT), jnp.float32)] * _NBUF
        + [pltpu.VMEM((_CHUNK,), jnp.int32)] * _NBUF
        + [
            pltpu.VMEM((_ZROWS, _FEAT), jnp.float32),
            pltpu.VMEM_SHARED((_ACLASS, _FEAT), jnp.float32),
        ]
        + [pltpu.SemaphoreType.DMA] * (2 * _NBUF)
    ),
)(_sc_segment_sum)


def _tc_partial(tgt_ref, rows_ref, o_ref):
    @pl.when(pl.program_id(0) == 0)
    def _():
        o_ref[...] = jnp.zeros_like(o_ref)

    tgt = tgt_ref[0, 0, :]
    cls = lax.broadcasted_iota(jnp.int32, (_NUM_CLASS, _TCBLK), 0)
    onehot_t = (cls == tgt[None, :]).astype(jnp.float32)
    o_ref[...] += jnp.dot(onehot_t, rows_ref[...],
                          preferred_element_type=jnp.float32)


def _combine(parts_ref, tc_ref, cs_ref, o_ref):
    o_ref[...] = (cs_ref[...] + tc_ref[...]
                  + parts_ref[0, :_NUM_CLASS] + parts_ref[1, :_NUM_CLASS])


def kernel(batch_samples, targets, idx, class_sums):
    del idx
    parts = _sc_call(batch_samples, targets)
    tgt_tc = targets[_S:].reshape(_T // _TCBLK, 1, _TCBLK)
    tc_part = pl.pallas_call(
        _tc_partial,
        grid=(_T // _TCBLK,),
        in_specs=[
            pl.BlockSpec((1, 1, _TCBLK), lambda i: (i, 0, 0)),
            pl.BlockSpec((_TCBLK, _FEAT), lambda i: (_S // _TCBLK + i, 0)),
        ],
        out_specs=pl.BlockSpec((_NUM_CLASS, _FEAT), lambda i: (0, 0)),
        out_shape=jax.ShapeDtypeStruct((_NUM_CLASS, _FEAT), jnp.float32),
    )(tgt_tc, batch_samples)
    return pl.pallas_call(
        _combine,
        out_shape=jax.ShapeDtypeStruct((_NUM_CLASS, _FEAT), jnp.float32),
    )(parts, tc_part, class_sums)


# speedup vs baseline: 1.9999x; 1.9999x over previous
import functools

import jax
import jax.numpy as jnp
from jax import lax
from jax.experimental import pallas as pl
from jax.experimental.pallas import tpu as pltpu
from jax.experimental.pallas import tpu_sc as plsc

_NUM_CLASS = 100
_FEAT = 128
_N = 320000
_NC = 2
_NS = 16
_NW = _NC * _NS
_ROWS_PER_W = _N // _NW
_CHUNK = 128
_NFULL = _ROWS_PER_W // _CHUNK
_TAIL = _ROWS_PER_W - _NFULL * _CHUNK
_ACLASS = 104
_ZROWS = 8
_NBUF = 4


def _sc_segment_sum(batch_hbm, tgt_hbm, parts_hbm,
                    rows_buf0, rows_buf1, rows_buf2, rows_buf3,
                    tgt_buf0, tgt_buf1, tgt_buf2, tgt_buf3,
                    rows_tail, tgt_tail, zero_buf, acc,
                    gsem0, gsem1, gsem2, gsem3,
                    ssem0, ssem1, ssem2, ssem3):
    cid = lax.axis_index("c")
    sid = lax.axis_index("s")
    wid = cid * _NS + sid
    base = wid * _ROWS_PER_W
    rows_bufs = (rows_buf0, rows_buf1, rows_buf2, rows_buf3)
    tgt_bufs = (tgt_buf0, tgt_buf1, tgt_buf2, tgt_buf3)
    gsems = (gsem0, gsem1, gsem2, gsem3)
    ssems = (ssem0, ssem1, ssem2, ssem3)

    zeros16 = jnp.zeros((16,), jnp.float32)

    def zero_body(i, carry):
        r = i // (_FEAT // 16)
        c = i % (_FEAT // 16)
        zero_buf[r, pl.ds(c * 16, 16)] = zeros16
        return carry

    @pl.when(sid < _ACLASS // _ZROWS)
    def _():
        lax.fori_loop(0, _ZROWS * (_FEAT // 16), zero_body, 0)
        pltpu.sync_copy(zero_buf, acc.at[pl.ds(sid * _ZROWS, _ZROWS)])

    plsc.subcore_barrier()

    def start_gather(ci, b):
        off = base + ci * _CHUNK
        pltpu.async_copy(tgt_hbm.at[pl.ds(off, _CHUNK)], tgt_bufs[b], gsems[b])
        pltpu.async_copy(batch_hbm.at[pl.ds(off, _CHUNK)], rows_bufs[b],
                         gsems[b])

    def wait_gather(ci, b):
        off = base + ci * _CHUNK
        pltpu.make_async_copy(tgt_hbm.at[pl.ds(off, _CHUNK)], tgt_bufs[b],
                              gsems[b]).wait()
        pltpu.make_async_copy(batch_hbm.at[pl.ds(off, _CHUNK)], rows_bufs[b],
                              gsems[b]).wait()

    def start_scatter(b):
        pltpu.async_copy(rows_bufs[b], acc.at[tgt_bufs[b]], ssems[b],
                         add=True)

    def wait_scatter(b):
        pltpu.make_async_copy(rows_bufs[b], acc.at[tgt_bufs[b]],
                              ssems[b]).wait()

    start_gather(0, 0)
    start_gather(1, 1)

    def chunk_body(i, carry):
        for b in range(_NBUF):
            ci = _NBUF * i + b

            @pl.when(ci < _NFULL)
            def _():
                @pl.when(jnp.logical_and(ci >= 2, ci + 2 < _NFULL))
                def _():
                    wait_scatter((b + 2) % _NBUF)

                @pl.when(ci + 2 < _NFULL)
                def _():
                    start_gather(ci + 2, (b + 2) % _NBUF)

                wait_gather(ci, b)
                start_scatter(b)
        return carry

    lax.fori_loop(0, (_NFULL + _NBUF - 1) // _NBUF, chunk_body, 0)

    for k in range(_NBUF):
        wait_scatter((_NFULL - _NBUF + k) % _NBUF)

    off = base + _NFULL * _CHUNK
    pltpu.sync_copy(tgt_hbm.at[pl.ds(off, _TAIL)], tgt_tail)
    pltpu.sync_copy(batch_hbm.at[pl.ds(off, _TAIL)], rows_tail)
    pltpu.sync_copy(rows_tail, acc.at[tgt_tail], add=True)

    plsc.subcore_barrier()

    @pl.when(sid < _ACLASS // _ZROWS)
    def _():
        pltpu.sync_copy(acc.at[pl.ds(sid * _ZROWS, _ZROWS)],
                        parts_hbm.at[cid, pl.ds(sid * _ZROWS, _ZROWS)])


_sc_call = functools.partial(
    pl.kernel,
    out_type=jax.ShapeDtypeStruct((_NC, _ACLASS, _FEAT), jnp.float32),
    mesh=plsc.VectorSubcoreMesh(core_axis_name="c", subcore_axis_name="s"),
    scratch_types=(
        [pltpu.VMEM((_CHUNK, _FEAT), jnp.float32)] * _NBUF
        + [pltpu.VMEM((_CHUNK,), jnp.int32)] * _NBUF
        + [
            pltpu.VMEM((_TAIL, _FEAT), jnp.float32),
            pltpu.VMEM((_TAIL,), jnp.int32),
            pltpu.VMEM((_ZROWS, _FEAT), jnp.float32),
            pltpu.VMEM_SHARED((_ACLASS, _FEAT), jnp.float32),
        ]
        + [pltpu.SemaphoreType.DMA] * (2 * _NBUF)
    ),
)(_sc_segment_sum)


def _combine(parts_ref, cs_ref, o_ref):
    o_ref[...] = (cs_ref[...]
                  + parts_ref[0, :_NUM_CLASS] + parts_ref[1, :_NUM_CLASS])


def kernel(batch_samples, targets, idx, class_sums):
    del idx
    parts = _sc_call(batch_samples, targets)
    return pl.pallas_call(
        _combine,
        out_shape=jax.ShapeDtypeStruct((_NUM_CLASS, _FEAT), jnp.float32),
    )(parts, class_sums)


# speedup vs baseline: 2.0303x; 1.0152x over previous
import functools

import jax
import jax.numpy as jnp
from jax import lax
from jax.experimental import pallas as pl
from jax.experimental.pallas import tpu as pltpu
from jax.experimental.pallas import tpu_sc as plsc

_NUM_CLASS = 100
_FEAT = 128
_N = 320000
_NC = 2
_NS = 16
_NW = _NC * _NS
_ROWS_PER_W = _N // _NW
_CHUNK = 128
_NFULL = _ROWS_PER_W // _CHUNK
_TAIL = _ROWS_PER_W - _NFULL * _CHUNK
_ACLASS = 104
_ZROWS = 8
_NBUF = 4


def _sc_segment_sum(batch_hbm, tgt_hbm, parts_hbm,
                    rows_buf0, rows_buf1, rows_buf2, rows_buf3,
                    tgt_buf0, tgt_buf1, tgt_buf2, tgt_buf3,
                    rows_tail, tgt_tail, zero_buf, acc,
                    gsem0, gsem1, gsem2, gsem3,
                    ssem0, ssem1, ssem2, ssem3, tsem):
    cid = lax.axis_index("c")
    sid = lax.axis_index("s")
    wid = cid * _NS + sid
    base = wid * _ROWS_PER_W
    rows_bufs = (rows_buf0, rows_buf1, rows_buf2, rows_buf3)
    tgt_bufs = (tgt_buf0, tgt_buf1, tgt_buf2, tgt_buf3)
    gsems = (gsem0, gsem1, gsem2, gsem3)
    ssems = (ssem0, ssem1, ssem2, ssem3)

    def start_gather(ci, b):
        off = base + ci * _CHUNK
        pltpu.async_copy(tgt_hbm.at[pl.ds(off, _CHUNK)], tgt_bufs[b], gsems[b])
        pltpu.async_copy(batch_hbm.at[pl.ds(off, _CHUNK)], rows_bufs[b],
                         gsems[b])

    toff = base + _NFULL * _CHUNK
    start_gather(0, 0)
    start_gather(1, 1)
    pltpu.async_copy(tgt_hbm.at[pl.ds(toff, _TAIL)], tgt_tail, tsem)
    pltpu.async_copy(batch_hbm.at[pl.ds(toff, _TAIL)], rows_tail, tsem)

    zeros16 = jnp.zeros((16,), jnp.float32)

    def zero_body(i, carry):
        r = i // (_FEAT // 16)
        c = i % (_FEAT // 16)
        zero_buf[r, pl.ds(c * 16, 16)] = zeros16
        return carry

    @pl.when(sid < _ACLASS // _ZROWS)
    def _():
        lax.fori_loop(0, _ZROWS * (_FEAT // 16), zero_body, 0)
        pltpu.sync_copy(zero_buf, acc.at[pl.ds(sid * _ZROWS, _ZROWS)])

    plsc.subcore_barrier()

    def wait_gather(ci, b):
        off = base + ci * _CHUNK
        pltpu.make_async_copy(tgt_hbm.at[pl.ds(off, _CHUNK)], tgt_bufs[b],
                              gsems[b]).wait()
        pltpu.make_async_copy(batch_hbm.at[pl.ds(off, _CHUNK)], rows_bufs[b],
                              gsems[b]).wait()

    def start_scatter(b):
        pltpu.async_copy(rows_bufs[b], acc.at[tgt_bufs[b]], ssems[b],
                         add=True)

    def wait_scatter(b):
        pltpu.make_async_copy(rows_bufs[b], acc.at[tgt_bufs[b]],
                              ssems[b]).wait()

    def chunk_body(i, carry):
        for b in range(_NBUF):
            ci = _NBUF * i + b

            @pl.when(ci < _NFULL)
            def _():
                @pl.when(jnp.logical_and(ci >= 2, ci + 2 < _NFULL))
                def _():
                    wait_scatter((b + 2) % _NBUF)

                @pl.when(ci + 2 < _NFULL)
                def _():
                    start_gather(ci + 2, (b + 2) % _NBUF)

                wait_gather(ci, b)
                start_scatter(b)
        return carry

    lax.fori_loop(0, (_NFULL + _NBUF - 1) // _NBUF, chunk_body, 0)

    for k in range(_NBUF):
        wait_scatter((_NFULL - _NBUF + k) % _NBUF)

    pltpu.make_async_copy(tgt_hbm.at[pl.ds(toff, _TAIL)], tgt_tail,
                          tsem).wait()
    pltpu.make_async_copy(batch_hbm.at[pl.ds(toff, _TAIL)], rows_tail,
                          tsem).wait()
    pltpu.sync_copy(rows_tail, acc.at[tgt_tail], add=True)

    plsc.subcore_barrier()

    @pl.when(sid < _ACLASS // _ZROWS)
    def _():
        pltpu.sync_copy(acc.at[pl.ds(sid * _ZROWS, _ZROWS)],
                        parts_hbm.at[cid, pl.ds(sid * _ZROWS, _ZROWS)])


_sc_call = functools.partial(
    pl.kernel,
    out_type=jax.ShapeDtypeStruct((_NC, _ACLASS, _FEAT), jnp.float32),
    mesh=plsc.VectorSubcoreMesh(core_axis_name="c", subcore_axis_name="s"),
    scratch_types=(
        [pltpu.VMEM((_CHUNK, _FEAT), jnp.float32)] * _NBUF
        + [pltpu.VMEM((_CHUNK,), jnp.int32)] * _NBUF
        + [
            pltpu.VMEM((_TAIL, _FEAT), jnp.float32),
            pltpu.VMEM((_TAIL,), jnp.int32),
            pltpu.VMEM((_ZROWS, _FEAT), jnp.float32),
            pltpu.VMEM_SHARED((_ACLASS, _FEAT), jnp.float32),
        ]
        + [pltpu.SemaphoreType.DMA] * (2 * _NBUF + 1)
    ),
)(_sc_segment_sum)


def _combine(parts_ref, cs_ref, o_ref):
    o_ref[...] = (cs_ref[...]
                  + parts_ref[0, :_NUM_CLASS] + parts_ref[1, :_NUM_CLASS])


def kernel(batch_samples, targets, idx, class_sums):
    del idx
    parts = _sc_call(batch_samples, targets)
    return pl.pallas_call(
        _combine,
        out_shape=jax.ShapeDtypeStruct((_NUM_CLASS, _FEAT), jnp.float32),
    )(parts, class_sums)
